# Initial kernel scaffold; baseline (speedup 1.0000x reference)
import functools

import jax
import jax.numpy as jnp
from jax import lax
from jax.experimental import pallas as pl
from jax.experimental.pallas import tpu as pltpu
from jax.experimental.pallas import tpu_sc as plsc

N = 10000
E = 160000
D = 256
HALF = 128
NS = 16
CH = 80
NCH = E // NS // CH
ROWS_PER_SUB = N // NS

_MESH = plsc.VectorSubcoreMesh(core_axis_name="c", subcore_axis_name="s")


@functools.partial(
    pl.kernel,
    out_type=(
        jax.ShapeDtypeStruct((N, 16), jnp.float32),
        jax.ShapeDtypeStruct((N, 16), jnp.float32),
    ),
    mesh=_MESH,
    scratch_types=[
        pltpu.VMEM((NCH, CH), jnp.int32),
        pltpu.VMEM((CH, 16), jnp.float32),
        pltpu.VMEM_SHARED((N, 16), jnp.float32),
    ],
)
def _deg_kernel(src_hbm, dst_hbm, z_hbm, ones_hbm,
                outdeg_hbm, indeg_hbm, idx_v, ones_v, deg_sh):
    c = lax.axis_index("c")
    s = lax.axis_index("s")
    pltpu.sync_copy(ones_hbm, ones_v)
    pltpu.sync_copy(z_hbm.at[pl.ds(s * ROWS_PER_SUB, ROWS_PER_SUB)],
                    deg_sh.at[pl.ds(s * ROWS_PER_SUB, ROWS_PER_SUB)])

    @pl.when(c == 0)
    def _():
        pltpu.sync_copy(src_hbm.at[pl.ds(s * NCH, NCH)], idx_v)

    @pl.when(c == 1)
    def _():
        pltpu.sync_copy(dst_hbm.at[pl.ds(s * NCH, NCH)], idx_v)

    plsc.subcore_barrier()

    @pl.loop(0, NCH)
    def _(j):
        pltpu.sync_copy(ones_v, deg_sh.at[idx_v.at[j]], add=True)

    plsc.subcore_barrier()

    @pl.when(c == 0)
    def _():
        pltpu.sync_copy(deg_sh.at[pl.ds(s * ROWS_PER_SUB, ROWS_PER_SUB)],
                        outdeg_hbm.at[pl.ds(s * ROWS_PER_SUB, ROWS_PER_SUB)])

    @pl.when(c == 1)
    def _():
        pltpu.sync_copy(deg_sh.at[pl.ds(s * ROWS_PER_SUB, ROWS_PER_SUB)],
                        indeg_hbm.at[pl.ds(s * ROWS_PER_SUB, ROWS_PER_SUB)])


@functools.partial(
    pl.kernel,
    out_type=(
        jax.ShapeDtypeStruct((N, HALF), jnp.float32),
        jax.ShapeDtypeStruct((N, HALF), jnp.float32),
    ),
    mesh=_MESH,
    scratch_types=[
        pltpu.VMEM((NCH, CH), jnp.int32),
        pltpu.VMEM((NCH, CH), jnp.int32),
        pltpu.VMEM((CH, HALF), jnp.float32),
        pltpu.VMEM_SHARED((N, HALF), jnp.float32),
        pltpu.SemaphoreType.DMA,
    ],
)
def _agg_kernel(h0_hbm, h1_hbm, src_hbm, dst_hbm, z_hbm,
                agg0_hbm, agg1_hbm, idxs_v, idxd_v, rows_v, agg_sh, sem):
    c = lax.axis_index("c")
    s = lax.axis_index("s")
    pltpu.sync_copy(src_hbm.at[pl.ds(s * NCH, NCH)], idxs_v)
    pltpu.sync_copy(dst_hbm.at[pl.ds(s * NCH, NCH)], idxd_v)
    pltpu.sync_copy(z_hbm.at[pl.ds(s * ROWS_PER_SUB, ROWS_PER_SUB)],
                    agg_sh.at[pl.ds(s * ROWS_PER_SUB, ROWS_PER_SUB)])
    plsc.subcore_barrier()

    def run(h_hbm, out_hbm):
        @pl.loop(0, NCH)
        def _(j):
            pltpu.async_copy(h_hbm.at[idxs_v.at[j]], rows_v, sem).wait()
            pltpu.sync_copy(rows_v, agg_sh.at[idxd_v.at[j]], add=True)

        plsc.subcore_barrier()
        pltpu.sync_copy(agg_sh.at[pl.ds(s * ROWS_PER_SUB, ROWS_PER_SUB)],
                        out_hbm.at[pl.ds(s * ROWS_PER_SUB, ROWS_PER_SUB)])

    @pl.when(c == 0)
    def _():
        run(h0_hbm, agg0_hbm)

    @pl.when(c == 1)
    def _():
        run(h1_hbm, agg1_hbm)


_BLK = 1000


def _scale_body(f_ref, d_ref, h0_ref, h1_ref):
    ns = lax.rsqrt(jnp.maximum(d_ref[:, 0:1], 1.0))
    h = f_ref[...] * ns
    h0_ref[...] = h[:, :HALF]
    h1_ref[...] = h[:, HALF:]


def _scale(feats, outdeg16):
    return pl.pallas_call(
        _scale_body,
        grid=(N // _BLK,),
        in_specs=[
            pl.BlockSpec((_BLK, D), lambda i: (i, 0)),
            pl.BlockSpec((_BLK, 16), lambda i: (i, 0)),
        ],
        out_specs=[
            pl.BlockSpec((_BLK, HALF), lambda i: (i, 0)),
            pl.BlockSpec((_BLK, HALF), lambda i: (i, 0)),
        ],
        out_shape=[
            jax.ShapeDtypeStruct((N, HALF), jnp.float32),
            jax.ShapeDtypeStruct((N, HALF), jnp.float32),
        ],
    )(feats, outdeg16)


def _final_body(a0_ref, a1_ref, d_ref, w_ref, b_ref, o_ref):
    nd = lax.rsqrt(jnp.maximum(d_ref[:, 0:1], 1.0))
    acc = jnp.dot(a0_ref[...] * nd, w_ref[:HALF, :],
                  preferred_element_type=jnp.float32,
                  precision=lax.Precision.HIGHEST)
    acc += jnp.dot(a1_ref[...] * nd, w_ref[HALF:, :],
                   preferred_element_type=jnp.float32,
                   precision=lax.Precision.HIGHEST)
    o_ref[...] = jnp.maximum(acc + b_ref[...], 0.0)


def _final(agg0, agg1, indeg16, W, b2d):
    return pl.pallas_call(
        _final_body,
        grid=(N // _BLK,),
        in_specs=[
            pl.BlockSpec((_BLK, HALF), lambda i: (i, 0)),
            pl.BlockSpec((_BLK, HALF), lambda i: (i, 0)),
            pl.BlockSpec((_BLK, 16), lambda i: (i, 0)),
            pl.BlockSpec((D, D), lambda i: (0, 0)),
            pl.BlockSpec((1, D), lambda i: (0, 0)),
        ],
        out_specs=pl.BlockSpec((_BLK, D), lambda i: (i, 0)),
        out_shape=jax.ShapeDtypeStruct((N, D), jnp.float32),
    )(agg0, agg1, indeg16, W, b2d)


def kernel(feats, edge_index, W, b):
    src = edge_index[0].reshape(E // CH, CH)
    dst = edge_index[1].reshape(E // CH, CH)
    zeros16 = jnp.zeros((N, 16), jnp.float32)
    ones16 = jnp.ones((CH, 16), jnp.float32)
    zeros128 = jnp.zeros((N, HALF), jnp.float32)
    outdeg16, indeg16 = _deg_kernel(src, dst, zeros16, ones16)
    h0, h1 = _scale(feats, outdeg16)
    agg0, agg1 = _agg_kernel(h0, h1, src, dst, zeros128)
    return _final(agg0, agg1, indeg16, W, b.reshape(1, D))


# speedup vs baseline: 3.7341x; 3.7341x over previous
import functools

import jax
import jax.numpy as jnp
from jax import lax
from jax.experimental import pallas as pl
from jax.experimental.pallas import tpu as pltpu
from jax.experimental.pallas import tpu_sc as plsc

N = 10000
E = 160000
D = 256
HALF = 128
NS = 16
CH = 80
NCH = E // NS // CH
ROWS_PER_SUB = N // NS

_MESH = plsc.VectorSubcoreMesh(core_axis_name="c", subcore_axis_name="s")


@functools.partial(
    pl.kernel,
    out_type=(
        jax.ShapeDtypeStruct((NS, ROWS_PER_SUB, HALF), jnp.float32),
        jax.ShapeDtypeStruct((NS, ROWS_PER_SUB, HALF), jnp.float32),
    ),
    mesh=_MESH,
    scratch_types=[
        pltpu.VMEM((NCH, CH), jnp.int32),
        pltpu.VMEM((CH, HALF), jnp.float32),
        pltpu.VMEM_SHARED((N, HALF), jnp.float32),
    ],
)
def _deg_kernel(src_hbm, dst_hbm, z_hbm, ones_hbm,
                outdeg_hbm, indeg_hbm, idx_v, ones_v, deg_sh):
    c = lax.axis_index("c")
    s = lax.axis_index("s")
    pltpu.sync_copy(ones_hbm, ones_v)
    pltpu.sync_copy(z_hbm.at[s],
                    deg_sh.at[pl.ds(s * ROWS_PER_SUB, ROWS_PER_SUB)])

    @pl.when(c == 0)
    def _():
        pltpu.sync_copy(src_hbm.at[s], idx_v)

    @pl.when(c == 1)
    def _():
        pltpu.sync_copy(dst_hbm.at[s], idx_v)

    plsc.subcore_barrier()

    @pl.loop(0, NCH)
    def _(j):
        pltpu.sync_copy(ones_v, deg_sh.at[idx_v.at[j]], add=True)

    plsc.subcore_barrier()

    @pl.when(c == 0)
    def _():
        pltpu.sync_copy(deg_sh.at[pl.ds(s * ROWS_PER_SUB, ROWS_PER_SUB)],
                        outdeg_hbm.at[s])

    @pl.when(c == 1)
    def _():
        pltpu.sync_copy(deg_sh.at[pl.ds(s * ROWS_PER_SUB, ROWS_PER_SUB)],
                        indeg_hbm.at[s])


@functools.partial(
    pl.kernel,
    out_type=(
        jax.ShapeDtypeStruct((NS, ROWS_PER_SUB, HALF), jnp.float32),
        jax.ShapeDtypeStruct((NS, ROWS_PER_SUB, HALF), jnp.float32),
    ),
    mesh=_MESH,
    scratch_types=[
        pltpu.VMEM((NCH, CH), jnp.int32),
        pltpu.VMEM((NCH, CH), jnp.int32),
        pltpu.VMEM((CH, HALF), jnp.float32),
        pltpu.VMEM_SHARED((N, HALF), jnp.float32),
        pltpu.SemaphoreType.DMA,
    ],
)
def _agg_kernel(h0_hbm, h1_hbm, src_hbm, dst_hbm, z_hbm,
                agg0_hbm, agg1_hbm, idxs_v, idxd_v, rows_v, agg_sh, sem):
    c = lax.axis_index("c")
    s = lax.axis_index("s")
    pltpu.sync_copy(src_hbm.at[s], idxs_v)
    pltpu.sync_copy(dst_hbm.at[s], idxd_v)
    pltpu.sync_copy(z_hbm.at[s],
                    agg_sh.at[pl.ds(s * ROWS_PER_SUB, ROWS_PER_SUB)])
    plsc.subcore_barrier()

    def run(h_hbm, out_hbm):
        @pl.loop(0, NCH)
        def _(j):
            pltpu.async_copy(h_hbm.at[idxs_v.at[j]], rows_v, sem).wait()
            pltpu.sync_copy(rows_v, agg_sh.at[idxd_v.at[j]], add=True)

        plsc.subcore_barrier()
        pltpu.sync_copy(agg_sh.at[pl.ds(s * ROWS_PER_SUB, ROWS_PER_SUB)],
                        out_hbm.at[s])

    @pl.when(c == 0)
    def _():
        run(h0_hbm, agg0_hbm)

    @pl.when(c == 1)
    def _():
        run(h1_hbm, agg1_hbm)


_BLK = 1000


def _scale_body(f_ref, d_ref, h0_ref, h1_ref):
    ns = lax.rsqrt(jnp.maximum(d_ref[:, 0:1], 1.0))
    h = f_ref[...] * ns
    h0_ref[...] = h[:, :HALF]
    h1_ref[...] = h[:, HALF:]


def _scale(feats, outdegw):
    return pl.pallas_call(
        _scale_body,
        grid=(N // _BLK,),
        in_specs=[
            pl.BlockSpec((_BLK, D), lambda i: (i, 0)),
            pl.BlockSpec((_BLK, HALF), lambda i: (i, 0)),
        ],
        out_specs=[
            pl.BlockSpec((_BLK, HALF), lambda i: (i, 0)),
            pl.BlockSpec((_BLK, HALF), lambda i: (i, 0)),
        ],
        out_shape=[
            jax.ShapeDtypeStruct((N, HALF), jnp.float32),
            jax.ShapeDtypeStruct((N, HALF), jnp.float32),
        ],
    )(feats, outdegw)


def _final_body(a0_ref, a1_ref, d_ref, w_ref, b_ref, o_ref):
    nd = lax.rsqrt(jnp.maximum(d_ref[:, 0:1], 1.0))
    acc = jnp.dot(a0_ref[...] * nd, w_ref[:HALF, :],
                  preferred_element_type=jnp.float32,
                  precision=lax.Precision.HIGHEST)
    acc += jnp.dot(a1_ref[...] * nd, w_ref[HALF:, :],
                   preferred_element_type=jnp.float32,
                   precision=lax.Precision.HIGHEST)
    o_ref[...] = jnp.maximum(acc + b_ref[...], 0.0)


def _final(agg0, agg1, indegw, W, b2d):
    return pl.pallas_call(
        _final_body,
        grid=(N // _BLK,),
        in_specs=[
            pl.BlockSpec((_BLK, HALF), lambda i: (i, 0)),
            pl.BlockSpec((_BLK, HALF), lambda i: (i, 0)),
            pl.BlockSpec((_BLK, HALF), lambda i: (i, 0)),
            pl.BlockSpec((D, D), lambda i: (0, 0)),
            pl.BlockSpec((1, D), lambda i: (0, 0)),
        ],
        out_specs=pl.BlockSpec((_BLK, D), lambda i: (i, 0)),
        out_shape=jax.ShapeDtypeStruct((N, D), jnp.float32),
    )(agg0, agg1, indegw, W, b2d)


def kernel(feats, edge_index, W, b):
    src = edge_index[0].reshape(NS, NCH, CH)
    dst = edge_index[1].reshape(NS, NCH, CH)
    zerosw = jnp.zeros((NS, ROWS_PER_SUB, HALF), jnp.float32)
    onesw = jnp.ones((CH, HALF), jnp.float32)
    outdegw, indegw = _deg_kernel(src, dst, zerosw, onesw)
    h0, h1 = _scale(feats, outdegw.reshape(N, HALF))
    agg0, agg1 = _agg_kernel(h0, h1, src, dst, zerosw)
    return _final(agg0.reshape(N, HALF), agg1.reshape(N, HALF),
                  indegw.reshape(N, HALF), W, b.reshape(1, D))


# speedup vs baseline: 4.5415x; 1.2162x over previous
import functools

import jax
import jax.numpy as jnp
from jax import lax
from jax.experimental import pallas as pl
from jax.experimental.pallas import tpu as pltpu
from jax.experimental.pallas import tpu_sc as plsc

N = 10000
E = 160000
D = 256
HALF = 128
NS = 16
NP = N + NS
CH = 128
EPS = E // NS
NCH = 80
PAD = NCH * CH - EPS
PH = 2
PCH = NCH // PH
RPS = NP // NS

_MESH = plsc.VectorSubcoreMesh(core_axis_name="c", subcore_axis_name="s")


@functools.partial(
    pl.kernel,
    out_type=(
        jax.ShapeDtypeStruct((NS, RPS, HALF), jnp.float32),
        jax.ShapeDtypeStruct((NS, RPS, HALF), jnp.float32),
    ),
    mesh=_MESH,
    scratch_types=[
        pltpu.VMEM((NCH, CH), jnp.int32),
        pltpu.VMEM((CH, HALF), jnp.float32),
        pltpu.VMEM_SHARED((NP, HALF), jnp.float32),
    ],
)
def _deg_kernel(src_hbm, dst_hbm, z_hbm, ones_hbm,
                outdeg_hbm, indeg_hbm, idx_v, ones_v, deg_sh):
    c = lax.axis_index("c")
    s = lax.axis_index("s")
    pltpu.sync_copy(ones_hbm, ones_v)
    pltpu.sync_copy(z_hbm.at[s], deg_sh.at[pl.ds(s * RPS, RPS)])

    @pl.when(c == 0)
    def _():
        pltpu.sync_copy(src_hbm.at[s], idx_v)

    @pl.when(c == 1)
    def _():
        pltpu.sync_copy(dst_hbm.at[s], idx_v)

    plsc.subcore_barrier()

    @pl.loop(0, NCH)
    def _(j):
        pltpu.sync_copy(ones_v, deg_sh.at[idx_v.at[j]], add=True)

    plsc.subcore_barrier()

    @pl.when(c == 0)
    def _():
        pltpu.sync_copy(deg_sh.at[pl.ds(s * RPS, RPS)], outdeg_hbm.at[s])

    @pl.when(c == 1)
    def _():
        pltpu.sync_copy(deg_sh.at[pl.ds(s * RPS, RPS)], indeg_hbm.at[s])


@functools.partial(
    pl.kernel,
    out_type=(
        jax.ShapeDtypeStruct((NS, RPS, HALF), jnp.float32),
        jax.ShapeDtypeStruct((NS, RPS, HALF), jnp.float32),
    ),
    mesh=_MESH,
    scratch_types=[
        pltpu.VMEM((PCH, CH), jnp.int32),
        pltpu.VMEM((PCH, CH), jnp.int32),
        pltpu.VMEM((CH, HALF), jnp.float32),
        pltpu.VMEM((CH, HALF), jnp.float32),
        pltpu.VMEM_SHARED((NP, HALF), jnp.float32),
        pltpu.SemaphoreType.DMA,
        pltpu.SemaphoreType.DMA,
    ],
)
def _agg_kernel(h0_hbm, h1_hbm, src_hbm, dst_hbm, z_hbm,
                agg0_hbm, agg1_hbm, idxs_v, idxd_v, rows0_v, rows1_v,
                agg_sh, gs0, gs1):
    c = lax.axis_index("c")
    s = lax.axis_index("s")
    pltpu.sync_copy(z_hbm.at[s], agg_sh.at[pl.ds(s * RPS, RPS)])
    plsc.subcore_barrier()

    def run(h_hbm, out_hbm):
        def gather(j, buf, sem):
            pltpu.async_copy(h_hbm.at[idxs_v.at[j]], buf, sem)

        def gwait(j, buf, sem):
            pltpu.make_async_copy(h_hbm.at[idxs_v.at[j]], buf, sem).wait()

        def scatter(j, buf, sem):
            pltpu.async_copy(buf, agg_sh.at[idxd_v.at[j]], sem,
                             add=True).wait()

        for ph in range(PH):
            pltpu.sync_copy(src_hbm.at[s].at[pl.ds(ph * PCH, PCH)], idxs_v)
            pltpu.sync_copy(dst_hbm.at[s].at[pl.ds(ph * PCH, PCH)], idxd_v)
            gather(0, rows0_v, gs0)

            @pl.loop(0, PCH // 2 - 1)
            def _(p):
                a = 1 + 2 * p
                gather(a, rows1_v, gs1)
                gwait(2 * p, rows0_v, gs0)
                scatter(2 * p, rows0_v, gs0)
                gather(a + 1, rows0_v, gs0)
                gwait(a, rows1_v, gs1)
                scatter(a, rows1_v, gs1)

            gather(PCH - 1, rows1_v, gs1)
            gwait(PCH - 2, rows0_v, gs0)
            scatter(PCH - 2, rows0_v, gs0)
            gwait(PCH - 1, rows1_v, gs1)
            scatter(PCH - 1, rows1_v, gs1)

        plsc.subcore_barrier()
        pltpu.sync_copy(agg_sh.at[pl.ds(s * RPS, RPS)], out_hbm.at[s])

    @pl.when(c == 0)
    def _():
        run(h0_hbm, agg0_hbm)

    @pl.when(c == 1)
    def _():
        run(h1_hbm, agg1_hbm)


_BLK = 1000


def _scale_body(f_ref, d_ref, h0_ref, h1_ref):
    ns = lax.rsqrt(jnp.maximum(d_ref[:, 0:1], 1.0))
    h = f_ref[...] * ns
    h0_ref[...] = h[:, :HALF]
    h1_ref[...] = h[:, HALF:]


def _scale(feats, outdegw):
    return pl.pallas_call(
        _scale_body,
        grid=(N // _BLK,),
        in_specs=[
            pl.BlockSpec((_BLK, D), lambda i: (i, 0)),
            pl.BlockSpec((_BLK, HALF), lambda i: (i, 0)),
        ],
        out_specs=[
            pl.BlockSpec((_BLK, HALF), lambda i: (i, 0)),
            pl.BlockSpec((_BLK, HALF), lambda i: (i, 0)),
        ],
        out_shape=[
            jax.ShapeDtypeStruct((N, HALF), jnp.float32),
            jax.ShapeDtypeStruct((N, HALF), jnp.float32),
        ],
    )(feats, outdegw)


def _final_body(a0_ref, a1_ref, d_ref, w_ref, b_ref, o_ref):
    nd = lax.rsqrt(jnp.maximum(d_ref[:, 0:1], 1.0))
    acc = jnp.dot(a0_ref[...] * nd, w_ref[:HALF, :],
                  preferred_element_type=jnp.float32,
                  precision=lax.Precision.HIGHEST)
    acc += jnp.dot(a1_ref[...] * nd, w_ref[HALF:, :],
                   preferred_element_type=jnp.float32,
                   precision=lax.Precision.HIGHEST)
    o_ref[...] = jnp.maximum(acc + b_ref[...], 0.0)


def _final(agg0, agg1, indegw, W, b2d):
    return pl.pallas_call(
        _final_body,
        grid=(N // _BLK,),
        in_specs=[
            pl.BlockSpec((_BLK, HALF), lambda i: (i, 0)),
            pl.BlockSpec((_BLK, HALF), lambda i: (i, 0)),
            pl.BlockSpec((_BLK, HALF), lambda i: (i, 0)),
            pl.BlockSpec((D, D), lambda i: (0, 0)),
            pl.BlockSpec((1, D), lambda i: (0, 0)),
        ],
        out_specs=pl.BlockSpec((_BLK, D), lambda i: (i, 0)),
        out_shape=jax.ShapeDtypeStruct((N, D), jnp.float32),
    )(agg0, agg1, indegw, W, b2d)


def kernel(feats, edge_index, W, b):
    trash = jnp.broadcast_to(N + jnp.arange(NS, dtype=jnp.int32)[:, None],
                             (NS, PAD))
    src = jnp.concatenate(
        [edge_index[0].reshape(NS, EPS), trash], axis=1).reshape(NS, NCH, CH)
    dst = jnp.concatenate(
        [edge_index[1].reshape(NS, EPS), trash], axis=1).reshape(NS, NCH, CH)
    zerosw = jnp.zeros((NS, RPS, HALF), jnp.float32)
    onesw = jnp.ones((CH, HALF), jnp.float32)
    outdegw, indegw = _deg_kernel(src, dst, zerosw, onesw)
    outdegw = outdegw.reshape(NP, HALF)[:N]
    indegw = indegw.reshape(NP, HALF)[:N]
    h0, h1 = _scale(feats, outdegw)
    h0 = jnp.pad(h0, ((0, NP - N), (0, 0)))
    h1 = jnp.pad(h1, ((0, NP - N), (0, 0)))
    agg0, agg1 = _agg_kernel(h0, h1, src, dst, zerosw)
    return _final(agg0.reshape(NP, HALF)[:N], agg1.reshape(NP, HALF)[:N],
                  indegw, W, b.reshape(1, D))


# speedup vs baseline: 5.7951x; 1.2760x over previous
import functools

import jax
import jax.numpy as jnp
from jax import lax
from jax.experimental import pallas as pl
from jax.experimental.pallas import tpu as pltpu
from jax.experimental.pallas import tpu_sc as plsc

N = 10000
E = 160000
D = 256
HALF = 128
NS = 16
NP = N + NS
CH = 128
EPS = E // NS
NCH = 80
PAD = NCH * CH - EPS
PH = 2
PCH = NCH // PH
RPS = NP // NS
RD = 80
RDS = RD // NS

_MESH = plsc.VectorSubcoreMesh(core_axis_name="c", subcore_axis_name="s")


@functools.partial(
    pl.kernel,
    out_type=(
        jax.ShapeDtypeStruct((NS, RD * 128), jnp.int32),
        jax.ShapeDtypeStruct((NS, RD * 128), jnp.int32),
    ),
    mesh=_MESH,
    scratch_types=[
        pltpu.VMEM((NCH, CH), jnp.int32),
        pltpu.VMEM((RD * 128,), jnp.int32),
    ],
    compiler_params=pltpu.CompilerParams(needs_layout_passes=False),
)
def _deg_kernel(src_hbm, dst_hbm, outdeg_hbm, indeg_hbm, idx_v, hist_v):
    c = lax.axis_index("c")
    s = lax.axis_index("s")

    @pl.when(c == 0)
    def _():
        pltpu.sync_copy(src_hbm.at[s], idx_v)

    @pl.when(c == 1)
    def _():
        pltpu.sync_copy(dst_hbm.at[s], idx_v)

    zeros16 = jnp.zeros((16,), jnp.int32)

    @pl.loop(0, RD * 8)
    def _(i):
        hist_v[pl.ds(i * 16, 16)] = zeros16

    ones16 = jnp.ones((16,), jnp.int32)

    @pl.loop(0, NCH)
    def _(j):
        @pl.loop(0, CH // 16)
        def _(k):
            idx16 = idx_v[j, pl.ds(k * 16, 16)]
            plsc.addupdate_scatter(hist_v, [idx16], ones16)

    @pl.when(c == 0)
    def _():
        pltpu.sync_copy(hist_v, outdeg_hbm.at[s])

    @pl.when(c == 1)
    def _():
        pltpu.sync_copy(hist_v, indeg_hbm.at[s])


def _degsum_body(op_ref, ip_ref, ns_ref, nd_ref):
    od = jnp.sum(op_ref[...], axis=0, keepdims=True).astype(jnp.float32)
    idg = jnp.sum(ip_ref[...], axis=0, keepdims=True).astype(jnp.float32)
    ns_ref[...] = lax.rsqrt(jnp.maximum(od, 1.0))
    nd_ref[...] = lax.rsqrt(jnp.maximum(idg, 1.0))


def _degsum(outp, inp):
    return pl.pallas_call(
        _degsum_body,
        out_shape=[
            jax.ShapeDtypeStruct((1, RD * 128), jnp.float32),
            jax.ShapeDtypeStruct((1, RD * 128), jnp.float32),
        ],
    )(outp, inp)


@functools.partial(
    pl.kernel,
    out_type=(
        jax.ShapeDtypeStruct((NS, RPS, HALF), jnp.float32),
        jax.ShapeDtypeStruct((NS, RPS, HALF), jnp.float32),
    ),
    mesh=_MESH,
    scratch_types=[
        pltpu.VMEM((PCH, CH), jnp.int32),
        pltpu.VMEM((PCH, CH), jnp.int32),
        pltpu.VMEM((CH, HALF), jnp.float32),
        pltpu.VMEM((CH, HALF), jnp.float32),
        pltpu.VMEM_SHARED((NP, HALF), jnp.float32),
        pltpu.SemaphoreType.DMA,
        pltpu.SemaphoreType.DMA,
    ],
)
def _agg_kernel(h0_hbm, h1_hbm, src_hbm, dst_hbm, z_hbm,
                agg0_hbm, agg1_hbm, idxs_v, idxd_v, rows0_v, rows1_v,
                agg_sh, gs0, gs1):
    c = lax.axis_index("c")
    s = lax.axis_index("s")
    pltpu.sync_copy(z_hbm.at[s], agg_sh.at[pl.ds(s * RPS, RPS)])
    plsc.subcore_barrier()

    def run(h_hbm, out_hbm):
        def gather(j, buf, sem):
            pltpu.async_copy(h_hbm.at[idxs_v.at[j]], buf, sem)

        def gwait(j, buf, sem):
            pltpu.make_async_copy(h_hbm.at[idxs_v.at[j]], buf, sem).wait()

        def scatter(j, buf, sem):
            pltpu.async_copy(buf, agg_sh.at[idxd_v.at[j]], sem,
                             add=True).wait()

        for ph in range(PH):
            pltpu.sync_copy(src_hbm.at[s].at[pl.ds(ph * PCH, PCH)], idxs_v)
            pltpu.sync_copy(dst_hbm.at[s].at[pl.ds(ph * PCH, PCH)], idxd_v)
            gather(0, rows0_v, gs0)

            @pl.loop(0, PCH // 2 - 1)
            def _(p):
                a = 1 + 2 * p
                gather(a, rows1_v, gs1)
                gwait(2 * p, rows0_v, gs0)
                scatter(2 * p, rows0_v, gs0)
                gather(a + 1, rows0_v, gs0)
                gwait(a, rows1_v, gs1)
                scatter(a, rows1_v, gs1)

            gather(PCH - 1, rows1_v, gs1)
            gwait(PCH - 2, rows0_v, gs0)
            scatter(PCH - 2, rows0_v, gs0)
            gwait(PCH - 1, rows1_v, gs1)
            scatter(PCH - 1, rows1_v, gs1)

        plsc.subcore_barrier()
        pltpu.sync_copy(agg_sh.at[pl.ds(s * RPS, RPS)], out_hbm.at[s])

    @pl.when(c == 0)
    def _():
        run(h0_hbm, agg0_hbm)

    @pl.when(c == 1)
    def _():
        run(h1_hbm, agg1_hbm)


_BLK = 1000


def _scale_body(f_ref, d_ref, h0_ref, h1_ref):
    h = f_ref[...] * d_ref[...]
    h0_ref[...] = h[:, :HALF]
    h1_ref[...] = h[:, HALF:]


def _scale(feats, outdegw):
    return pl.pallas_call(
        _scale_body,
        grid=(N // _BLK,),
        in_specs=[
            pl.BlockSpec((_BLK, D), lambda i: (i, 0)),
            pl.BlockSpec((_BLK, 1), lambda i: (i, 0)),
        ],
        out_specs=[
            pl.BlockSpec((_BLK, HALF), lambda i: (i, 0)),
            pl.BlockSpec((_BLK, HALF), lambda i: (i, 0)),
        ],
        out_shape=[
            jax.ShapeDtypeStruct((N, HALF), jnp.float32),
            jax.ShapeDtypeStruct((N, HALF), jnp.float32),
        ],
    )(feats, outdegw)


def _final_body(a0_ref, a1_ref, d_ref, w_ref, b_ref, o_ref):
    nd = d_ref[...]
    acc = jnp.dot(a0_ref[...] * nd, w_ref[:HALF, :],
                  preferred_element_type=jnp.float32,
                  precision=lax.Precision.HIGHEST)
    acc += jnp.dot(a1_ref[...] * nd, w_ref[HALF:, :],
                   preferred_element_type=jnp.float32,
                   precision=lax.Precision.HIGHEST)
    o_ref[...] = jnp.maximum(acc + b_ref[...], 0.0)


def _final(agg0, agg1, indegw, W, b2d):
    return pl.pallas_call(
        _final_body,
        grid=(N // _BLK,),
        in_specs=[
            pl.BlockSpec((_BLK, HALF), lambda i: (i, 0)),
            pl.BlockSpec((_BLK, HALF), lambda i: (i, 0)),
            pl.BlockSpec((_BLK, 1), lambda i: (i, 0)),
            pl.BlockSpec((D, D), lambda i: (0, 0)),
            pl.BlockSpec((1, D), lambda i: (0, 0)),
        ],
        out_specs=pl.BlockSpec((_BLK, D), lambda i: (i, 0)),
        out_shape=jax.ShapeDtypeStruct((N, D), jnp.float32),
    )(agg0, agg1, indegw, W, b2d)


def kernel(feats, edge_index, W, b):
    trash = jnp.broadcast_to(N + jnp.arange(NS, dtype=jnp.int32)[:, None],
                             (NS, PAD))
    src = jnp.concatenate(
        [edge_index[0].reshape(NS, EPS), trash], axis=1).reshape(NS, NCH, CH)
    dst = jnp.concatenate(
        [edge_index[1].reshape(NS, EPS), trash], axis=1).reshape(NS, NCH, CH)
    zerosw = jnp.zeros((NS, RPS, HALF), jnp.float32)
    outp, inp = _deg_kernel(src, dst)
    outdegw, indegw = _degsum(outp, inp)
    outdegw = outdegw.reshape(RD * 128)[:N].reshape(N, 1)
    indegw = indegw.reshape(RD * 128)[:N].reshape(N, 1)
    h0, h1 = _scale(feats, outdegw)
    h0 = jnp.pad(h0, ((0, NP - N), (0, 0)))
    h1 = jnp.pad(h1, ((0, NP - N), (0, 0)))
    agg0, agg1 = _agg_kernel(h0, h1, src, dst, zerosw)
    return _final(agg0.reshape(NP, HALF)[:N], agg1.reshape(NP, HALF)[:N],
                  indegw, W, b.reshape(1, D))


# speedup vs baseline: 5.8857x; 1.0156x over previous
import functools

import jax
import jax.numpy as jnp
from jax import lax
from jax.experimental import pallas as pl
from jax.experimental.pallas import tpu as pltpu
from jax.experimental.pallas import tpu_sc as plsc

N = 10000
E = 160000
D = 256
HALF = 128
NS = 16
NP = N + NS
CH = 128
EPS = E // NS
NCH = 80
PAD = NCH * CH - EPS
PH = 2
PCH = NCH // PH
RPS = NP // NS
RD = 80
RDS = RD // NS

_MESH = plsc.VectorSubcoreMesh(core_axis_name="c", subcore_axis_name="s")


@functools.partial(
    pl.kernel,
    out_type=(
        jax.ShapeDtypeStruct((NS, RD * 128), jnp.int32),
        jax.ShapeDtypeStruct((NS, RD * 128), jnp.int32),
    ),
    mesh=_MESH,
    scratch_types=[
        pltpu.VMEM((NCH, CH), jnp.int32),
        pltpu.VMEM((RD * 128,), jnp.int32),
    ],
    compiler_params=pltpu.CompilerParams(needs_layout_passes=False),
)
def _deg_kernel(src_hbm, dst_hbm, outdeg_hbm, indeg_hbm, idx_v, hist_v):
    c = lax.axis_index("c")
    s = lax.axis_index("s")

    @pl.when(c == 0)
    def _():
        pltpu.sync_copy(src_hbm.at[s], idx_v)

    @pl.when(c == 1)
    def _():
        pltpu.sync_copy(dst_hbm.at[s], idx_v)

    zeros16 = jnp.zeros((16,), jnp.int32)

    @pl.loop(0, RD * 8)
    def _(i):
        hist_v[pl.ds(i * 16, 16)] = zeros16

    ones16 = jnp.ones((16,), jnp.int32)

    @pl.loop(0, NCH)
    def _(j):
        @pl.loop(0, CH // 16)
        def _(k):
            idx16 = idx_v[j, pl.ds(k * 16, 16)]
            plsc.addupdate_scatter(hist_v, [idx16], ones16)

    @pl.when(c == 0)
    def _():
        pltpu.sync_copy(hist_v, outdeg_hbm.at[s])

    @pl.when(c == 1)
    def _():
        pltpu.sync_copy(hist_v, indeg_hbm.at[s])


def _degsum_body(op_ref, ip_ref, ns_ref, nd_ref):
    od = jnp.sum(op_ref[...], axis=0, keepdims=True).astype(jnp.float32)
    idg = jnp.sum(ip_ref[...], axis=0, keepdims=True).astype(jnp.float32)
    ns_ref[...] = lax.rsqrt(jnp.maximum(od, 1.0))
    nd_ref[...] = lax.rsqrt(jnp.maximum(idg, 1.0))


def _degsum(outp, inp):
    return pl.pallas_call(
        _degsum_body,
        out_shape=[
            jax.ShapeDtypeStruct((1, RD * 128), jnp.float32),
            jax.ShapeDtypeStruct((1, RD * 128), jnp.float32),
        ],
    )(outp, inp)


@functools.partial(
    pl.kernel,
    out_type=(
        jax.ShapeDtypeStruct((NS, RPS, HALF), jnp.float32),
        jax.ShapeDtypeStruct((NS, RPS, HALF), jnp.float32),
    ),
    mesh=_MESH,
    scratch_types=[
        pltpu.VMEM((PCH, CH), jnp.int32),
        pltpu.VMEM((PCH, CH), jnp.int32),
        pltpu.VMEM((CH, HALF), jnp.float32),
        pltpu.VMEM((CH, HALF), jnp.float32),
        pltpu.VMEM_SHARED((NP, HALF), jnp.float32),
        pltpu.SemaphoreType.DMA,
        pltpu.SemaphoreType.DMA,
    ],
)
def _agg_kernel(h0_hbm, h1_hbm, src_hbm, dst_hbm, z_hbm,
                agg0_hbm, agg1_hbm, idxs_v, idxd_v, rows0_v, rows1_v,
                agg_sh, gs0, gs1):
    c = lax.axis_index("c")
    s = lax.axis_index("s")
    pltpu.sync_copy(z_hbm.at[s], agg_sh.at[pl.ds(s * RPS, RPS)])
    plsc.subcore_barrier()

    def run(h_hbm, out_hbm):
        def gather(j, buf, sem):
            pltpu.async_copy(h_hbm.at[idxs_v.at[j]], buf, sem)

        def gwait(j, buf, sem):
            pltpu.make_async_copy(h_hbm.at[idxs_v.at[j]], buf, sem).wait()

        def scatter(j, buf, sem):
            pltpu.async_copy(buf, agg_sh.at[idxd_v.at[j]], sem,
                             add=True).wait()

        for ph in range(PH):
            pltpu.sync_copy(src_hbm.at[s].at[pl.ds(ph * PCH, PCH)], idxs_v)
            pltpu.sync_copy(dst_hbm.at[s].at[pl.ds(ph * PCH, PCH)], idxd_v)
            gather(0, rows0_v, gs0)

            @pl.loop(0, PCH // 2 - 1)
            def _(p):
                a = 1 + 2 * p
                gather(a, rows1_v, gs1)
                gwait(2 * p, rows0_v, gs0)
                scatter(2 * p, rows0_v, gs0)
                gather(a + 1, rows0_v, gs0)
                gwait(a, rows1_v, gs1)
                scatter(a, rows1_v, gs1)

            gather(PCH - 1, rows1_v, gs1)
            gwait(PCH - 2, rows0_v, gs0)
            scatter(PCH - 2, rows0_v, gs0)
            gwait(PCH - 1, rows1_v, gs1)
            scatter(PCH - 1, rows1_v, gs1)

        plsc.subcore_barrier()
        pltpu.sync_copy(agg_sh.at[pl.ds(s * RPS, RPS)], out_hbm.at[s])

    @pl.when(c == 0)
    def _():
        run(h0_hbm, agg0_hbm)

    @pl.when(c == 1)
    def _():
        run(h1_hbm, agg1_hbm)


_BLK = 1000


def _matmul_body(f_ref, w_ref, y_ref):
    y_ref[...] = jnp.dot(f_ref[...], w_ref[...],
                         preferred_element_type=jnp.float32,
                         precision=lax.Precision.HIGHEST)


def _matmul(feats, W):
    return pl.pallas_call(
        _matmul_body,
        grid=(N // _BLK,),
        in_specs=[
            pl.BlockSpec((_BLK, D), lambda i: (i, 0)),
            pl.BlockSpec((D, D), lambda i: (0, 0)),
        ],
        out_specs=pl.BlockSpec((_BLK, D), lambda i: (i, 0)),
        out_shape=jax.ShapeDtypeStruct((N, D), jnp.float32),
    )(feats, W)


def _scale_body(f_ref, d_ref, h0_ref, h1_ref):
    h = f_ref[...] * d_ref[...]
    h0_ref[...] = h[:, :HALF]
    h1_ref[...] = h[:, HALF:]


def _scale(feats, outdegw):
    return pl.pallas_call(
        _scale_body,
        grid=(N // _BLK,),
        in_specs=[
            pl.BlockSpec((_BLK, D), lambda i: (i, 0)),
            pl.BlockSpec((_BLK, 1), lambda i: (i, 0)),
        ],
        out_specs=[
            pl.BlockSpec((_BLK, HALF), lambda i: (i, 0)),
            pl.BlockSpec((_BLK, HALF), lambda i: (i, 0)),
        ],
        out_shape=[
            jax.ShapeDtypeStruct((N, HALF), jnp.float32),
            jax.ShapeDtypeStruct((N, HALF), jnp.float32),
        ],
    )(feats, outdegw)


def _final_body(a0_ref, a1_ref, d_ref, b_ref, o_ref):
    nd = d_ref[...]
    o_ref[:, :HALF] = jnp.maximum(a0_ref[...] * nd + b_ref[:, :HALF], 0.0)
    o_ref[:, HALF:] = jnp.maximum(a1_ref[...] * nd + b_ref[:, HALF:], 0.0)


def _final(agg0, agg1, indegw, b2d):
    return pl.pallas_call(
        _final_body,
        grid=(N // _BLK,),
        in_specs=[
            pl.BlockSpec((_BLK, HALF), lambda i: (i, 0)),
            pl.BlockSpec((_BLK, HALF), lambda i: (i, 0)),
            pl.BlockSpec((_BLK, 1), lambda i: (i, 0)),
            pl.BlockSpec((1, D), lambda i: (0, 0)),
        ],
        out_specs=pl.BlockSpec((_BLK, D), lambda i: (i, 0)),
        out_shape=jax.ShapeDtypeStruct((N, D), jnp.float32),
    )(agg0, agg1, indegw, b2d)


def kernel(feats, edge_index, W, b):
    trash = jnp.broadcast_to(N + jnp.arange(NS, dtype=jnp.int32)[:, None],
                             (NS, PAD))
    src = jnp.concatenate(
        [edge_index[0].reshape(NS, EPS), trash], axis=1).reshape(NS, NCH, CH)
    dst = jnp.concatenate(
        [edge_index[1].reshape(NS, EPS), trash], axis=1).reshape(NS, NCH, CH)
    zerosw = jnp.zeros((NS, RPS, HALF), jnp.float32)
    y = _matmul(feats, W)
    outp, inp = _deg_kernel(src, dst)
    outdegw, indegw = _degsum(outp, inp)
    outdegw = outdegw.reshape(RD * 128)[:N].reshape(N, 1)
    indegw = indegw.reshape(RD * 128)[:N].reshape(N, 1)
    h0, h1 = _scale(y, outdegw)
    h0 = jnp.pad(h0, ((0, NP - N), (0, 0)))
    h1 = jnp.pad(h1, ((0, NP - N), (0, 0)))
    agg0, agg1 = _agg_kernel(h0, h1, src, dst, zerosw)
    return _final(agg0.reshape(NP, HALF)[:N], agg1.reshape(NP, HALF)[:N],
                  indegw, b.reshape(1, D))
